# baseline (speedup 1.0000x reference)
import jax
import jax.numpy as jnp
from jax.experimental import pallas as pl
from jax.experimental.pallas import tpu as pltpu

_CR = 1024
_NBUF = 8
_D = 4


def _copy_body(x_hbm, o_hbm, *args):
    nchunks = x_hbm.shape[0] // _CR
    bufs = args[:_NBUF]
    sems = args[_NBUF:]
    isems = sems[:_NBUF]
    osems = sems[_NBUF:]

    def in_copy(c):
        b = c % _NBUF
        return pltpu.make_async_copy(
            x_hbm.at[pl.ds(c * _CR, _CR)], bufs[b], isems[b]
        )

    def out_copy(c):
        b = c % _NBUF
        return pltpu.make_async_copy(
            bufs[b], o_hbm.at[pl.ds(c * _CR, _CR)], osems[b]
        )

    for c in range(_D):
        in_copy(c).start()
    for c in range(nchunks):
        in_copy(c).wait()
        out_copy(c).start()
        if c >= _NBUF - _D:
            out_copy(c - (_NBUF - _D)).wait()
        if c + _D < nchunks:
            in_copy(c + _D).start()
    for c in range(nchunks - (_NBUF - _D), nchunks):
        out_copy(c).wait()


def kernel(stack):
    n, s, d = stack.shape
    x = stack.transpose(0, 2, 1).reshape(n * d, s)
    rows = n * d
    y = pl.pallas_call(
        _copy_body,
        in_specs=[pl.BlockSpec(memory_space=pl.ANY)],
        out_specs=pl.BlockSpec(memory_space=pl.ANY),
        out_shape=jax.ShapeDtypeStruct((rows, s), stack.dtype),
        scratch_shapes=[pltpu.VMEM((_CR, 1024), jnp.float32)] * _NBUF
        + [pltpu.SemaphoreType.DMA] * (2 * _NBUF),
    )(x)
    return y.reshape(n, d, s).transpose(0, 2, 1)
